# baseline (speedup 1.0000x reference)
import jax
import jax.numpy as jnp
from jax.experimental import pallas as pl
from jax.experimental.pallas import tpu as pltpu

FEATS_ = 128
K_ = 50000
CHUNK_ = 12500
NCHUNK_ = K_ // CHUNK_
NBUF_ = 3


def _manual_kernel(x_hbm, w_ref, o_hbm, xbuf, ybuf, insem, outsem):
    w = w_ref[...]
    inv_norm = jax.lax.rsqrt(jnp.sum(w * w))

    def in_copy(j, slot):
        return pltpu.make_async_copy(
            x_hbm.at[pl.ds(j * CHUNK_, CHUNK_), :], xbuf.at[slot],
            insem.at[slot])

    def out_copy(j, slot):
        return pltpu.make_async_copy(
            ybuf.at[slot], o_hbm.at[pl.ds(j * CHUNK_, CHUNK_), :],
            outsem.at[slot])

    for j in range(min(NBUF_, NCHUNK_)):
        in_copy(j, j % NBUF_).start()

    for j in range(NCHUNK_):
        slot = j % NBUF_
        in_copy(j, slot).wait()
        if j >= NBUF_:
            out_copy(j - NBUF_, slot).wait()
        x = xbuf[slot]
        s = jnp.dot(x, w, preferred_element_type=jnp.float32) * inv_norm
        ybuf[slot] = x * jnp.tanh(s)
        out_copy(j, slot).start()
        if j + NBUF_ < NCHUNK_:
            in_copy(j + NBUF_, slot).start()

    for j in range(max(NCHUNK_ - NBUF_, 0), NCHUNK_):
        out_copy(j, j % NBUF_).wait()


def kernel(node_embs, mask, scorer):
    del mask
    out = pl.pallas_call(
        _manual_kernel,
        in_specs=[
            pl.BlockSpec(memory_space=pl.ANY),
            pl.BlockSpec(memory_space=pltpu.VMEM),
        ],
        out_specs=pl.BlockSpec(memory_space=pl.ANY),
        out_shape=jax.ShapeDtypeStruct((K_, FEATS_), jnp.float32),
        compiler_params=pltpu.CompilerParams(vmem_limit_bytes=120*1024*1024),
        scratch_shapes=[
            pltpu.VMEM((NBUF_, CHUNK_, FEATS_), jnp.float32),
            pltpu.VMEM((NBUF_, CHUNK_, FEATS_), jnp.float32),
            pltpu.SemaphoreType.DMA((NBUF_,)),
            pltpu.SemaphoreType.DMA((NBUF_,)),
        ],
    )(node_embs, scorer)
    return out.T


# speedup vs baseline: 1.2129x; 1.2129x over previous
import jax
import jax.numpy as jnp
from jax.experimental import pallas as pl
from jax.experimental.pallas import tpu as pltpu

FEATS_ = 128
K_ = 50000
BLOCK_ = 25000


def _scale_kernel(x_ref, w_ref, o_ref):
    x = x_ref[...]
    w = w_ref[...]
    inv_norm = jax.lax.rsqrt(jnp.sum(w * w))
    s = jnp.dot(x, w, preferred_element_type=jnp.float32) * inv_norm
    o_ref[...] = x * jnp.tanh(s)


def kernel(node_embs, mask, scorer):
    del mask
    n_blocks = pl.cdiv(K_, BLOCK_)
    out = pl.pallas_call(
        _scale_kernel,
        grid=(n_blocks,),
        in_specs=[
            pl.BlockSpec((BLOCK_, FEATS_), lambda i: (i, 0)),
            pl.BlockSpec((FEATS_, 1), lambda i: (0, 0)),
        ],
        out_specs=pl.BlockSpec((BLOCK_, FEATS_), lambda i: (i, 0)),
        out_shape=jax.ShapeDtypeStruct((K_, FEATS_), jnp.float32),
        compiler_params=pltpu.CompilerParams(
            vmem_limit_bytes=120 * 1024 * 1024),
    )(node_embs, scorer)
    return out.T
